# baseline (speedup 1.0000x reference)
import functools

import jax
import jax.numpy as jnp
from jax import lax
from jax.experimental import pallas as pl
from jax.experimental.pallas import tpu as pltpu
from jax.experimental.pallas import tpu_sc as plsc

N = 10000
D = 128
G = 128
O = 64
E = 320000

NC = 2
NS = 16
CHUNK = 96
CPT = 108
NBUF = 3
EPAD = NC * NS * CPT * CHUNK
NPAD = 10240
RPT = NPAD // NS

_mesh = plsc.VectorSubcoreMesh(core_axis_name="c", subcore_axis_name="s")


def _zero_vmem2d(ref, rows, cols):
    def body(i, _):
        for j in range(cols // 16):
            ref[i, pl.ds(j * 16, 16)] = jnp.zeros((16,), jnp.float32)
        return 0
    lax.fori_loop(0, rows, body, 0)


def _unpack(comb, j, iu, dst_only=False):
    for k in range(CHUNK // 16):
        v = comb[pl.ds(j * CHUNK + k * 16, 16)]
        if not dst_only:
            iu[0, pl.ds(k * 16, 16)] = v & jnp.int32(0xFFFF)
        iu[1, pl.ds(k * 16, 16)] = lax.shift_right_logical(v, 16)


@functools.partial(
    pl.kernel,
    out_type=jax.ShapeDtypeStruct((NC, NPAD), jnp.float32),
    mesh=_mesh,
    scratch_types=[
        pltpu.VMEM_SHARED((NPAD,), jnp.float32),
        pltpu.VMEM((CPT * CHUNK,), jnp.int32),
        pltpu.VMEM((2, CHUNK), jnp.int32),
        pltpu.VMEM((CHUNK,), jnp.float32),
        pltpu.VMEM((RPT,), jnp.float32),
    ],
)
def _deg_kernel(comb_t, degp, deg_sh, comb_v, iu, ones_v, zv):
    c = lax.axis_index("c")
    s = lax.axis_index("s")
    pltpu.sync_copy(comb_t.at[c, s], comb_v)

    def zbody(i, _):
        zv[pl.ds(i * 16, 16)] = jnp.zeros((16,), jnp.float32)
        return 0
    lax.fori_loop(0, RPT // 16, zbody, 0)

    def obody(i, _):
        ones_v[pl.ds(i * 16, 16)] = jnp.ones((16,), jnp.float32)
        return 0
    lax.fori_loop(0, CHUNK // 16, obody, 0)

    pltpu.sync_copy(zv, deg_sh.at[pl.ds(s * RPT, RPT)])
    plsc.subcore_barrier()

    def sbody(j, _):
        _unpack(comb_v, j, iu, dst_only=True)
        pltpu.sync_copy(ones_v, deg_sh.at[iu.at[1]], add=True)
        return 0
    lax.fori_loop(0, CPT, sbody, 0)

    plsc.subcore_barrier()
    pltpu.sync_copy(deg_sh.at[pl.ds(s * RPT, RPT)], degp.at[c, pl.ds(s * RPT, RPT)])


@functools.partial(
    pl.kernel,
    out_type=jax.ShapeDtypeStruct((NC, NPAD, D), jnp.float32),
    mesh=_mesh,
    scratch_types=[
        pltpu.VMEM_SHARED((NPAD, D), jnp.float32),
        pltpu.VMEM((CPT * CHUNK,), jnp.int32),
        pltpu.VMEM((2, CHUNK), jnp.int32),
        pltpu.VMEM((2, CHUNK), jnp.int32),
        pltpu.VMEM((2, CHUNK), jnp.int32),
        pltpu.VMEM((CHUNK, D), jnp.float32),
        pltpu.VMEM((CHUNK, D), jnp.float32),
        pltpu.VMEM((CHUNK, D), jnp.float32),
        pltpu.SemaphoreType.DMA,
        pltpu.SemaphoreType.DMA,
        pltpu.SemaphoreType.DMA,
        pltpu.SemaphoreType.DMA,
        pltpu.SemaphoreType.DMA,
        pltpu.SemaphoreType.DMA,
    ],
)
def _edge_kernel(xws, comb_t, accp,
                 acc_sh, comb_v, iu0, iu1, iu2, rows0, rows1, rows2,
                 gsem0, gsem1, gsem2, ssem0, ssem1, ssem2):
    c = lax.axis_index("c")
    s = lax.axis_index("s")
    iu = (iu0, iu1, iu2)
    rows = (rows0, rows1, rows2)
    gsem = (gsem0, gsem1, gsem2)
    ssem = (ssem0, ssem1, ssem2)
    pltpu.sync_copy(comb_t.at[c, s], comb_v)

    _zero_vmem2d(rows0, 64, D)
    for r in range(RPT // 64):
        pltpu.sync_copy(rows0.at[pl.ds(0, 64)],
                        acc_sh.at[pl.ds(s * RPT + r * 64, 64)])
    plsc.subcore_barrier()

    def _drain(sem, buf):
        pltpu.make_async_copy(xws.at[pl.ds(0, CHUNK)], buf, sem).wait()

    def body(i, _):
        base = NBUF * i
        for q in range(NBUF):
            @pl.when(i > 0)
            def _():
                _drain(ssem[q], rows[q])
            _unpack(comb_v, base + q, iu[q])
            pltpu.async_copy(xws.at[iu[q].at[0]], rows[q], gsem[q])
        for q in range(NBUF):
            _drain(gsem[q], rows[q])
            pltpu.async_copy(rows[q], acc_sh.at[iu[q].at[1]], ssem[q], add=True)
        return 0

    lax.fori_loop(0, CPT // NBUF, body, 0)
    for q in range(NBUF):
        _drain(ssem[q], rows[q])

    plsc.subcore_barrier()
    pltpu.sync_copy(acc_sh.at[pl.ds(s * RPT, RPT)],
                    accp.at[c, pl.ds(s * RPT, RPT)])


BLK = 1024
_NBLK = NPAD // BLK


def _pre_body(x_ref, w_ref, d0_ref, d1_ref, xws_ref, dinv_ref):
    dinv = lax.rsqrt(d0_ref[...] + d1_ref[...] + 1.0)
    xw = jnp.dot(x_ref[...], w_ref[...], preferred_element_type=jnp.float32)
    xws_ref[...] = xw * dinv
    dinv_ref[...] = dinv


def _mid_body(a0_ref, a1_ref, xws_ref, dinv_ref, w_ref, out_ref):
    dinv = dinv_ref[...]
    h = jnp.maximum((a0_ref[...] + a1_ref[...] + xws_ref[...]) * dinv, 0.0)
    out_ref[...] = jnp.dot(h, w_ref[...], preferred_element_type=jnp.float32) * dinv


def _pool_body(a0_ref, a1_ref, xws_ref, dinv_ref, b_ref, out_ref, pool_acc):
    i = pl.program_id(0)
    h = jnp.maximum((a0_ref[...] + a1_ref[...] + xws_ref[...]) * dinv_ref[...], 0.0)
    oh = (b_ref[...] == lax.broadcasted_iota(jnp.int32, (1, G), 1)).astype(jnp.float32)
    p = lax.dot_general(oh, h, (((0,), (0,)), ((), ())),
                        preferred_element_type=jnp.float32, precision=lax.Precision.HIGHEST)

    @pl.when(i == 0)
    def _():
        pool_acc[...] = p

    @pl.when(i > 0)
    def _():
        pool_acc[...] = pool_acc[...] + p

    @pl.when(i == _NBLK - 1)
    def _():
        out_ref[...] = pool_acc[...]


def _mlp_body(pool_ref, b_ref, w1_ref, b1_ref, g_ref, be_ref, w2_ref, b2_ref,
              log_ref, sig_ref, last_ref):
    eq = (b_ref[...] == lax.broadcasted_iota(jnp.int32, (1, G), 1)).astype(jnp.float32)
    cnt = lax.dot_general(eq, jnp.ones((NPAD, 1), jnp.float32),
                          (((0,), (0,)), ((), ())),
                          preferred_element_type=jnp.float32, precision=lax.Precision.HIGHEST)
    g = pool_ref[...] / jnp.maximum(cnt, 1.0)
    m = jnp.dot(g, w1_ref[...], preferred_element_type=jnp.float32) + b1_ref[...]
    mu = jnp.mean(m, axis=0, keepdims=True)
    var = jnp.mean((m - mu) ** 2, axis=0, keepdims=True)
    m = (m - mu) / jnp.sqrt(var + 1e-5) * g_ref[...] + be_ref[...]
    m = jnp.maximum(m, 0.0)
    out = jnp.dot(m, w2_ref[...], preferred_element_type=jnp.float32) + b2_ref[...]
    xmax = jnp.max(out, axis=-1, keepdims=True)
    ex = jnp.exp(out - xmax)
    lse = jnp.log(jnp.sum(ex, axis=-1, keepdims=True)) + xmax
    log_ref[...] = out - lse
    sig_ref[...] = 1.0 / (1.0 + jnp.exp(-out))
    last_ref[...] = out


def _blk(i):
    return (i, 0)


def _fix(i):
    return (0, 0)


_f32 = jnp.float32


def _pre_call(x_pad, W0, d0, d1):
    return pl.pallas_call(
        _pre_body,
        grid=(_NBLK,),
        in_specs=[
            pl.BlockSpec((BLK, D), _blk),
            pl.BlockSpec((D, D), _fix),
            pl.BlockSpec((BLK, 1), _blk),
            pl.BlockSpec((BLK, 1), _blk),
        ],
        out_specs=[pl.BlockSpec((BLK, D), _blk), pl.BlockSpec((BLK, 1), _blk)],
        out_shape=[jax.ShapeDtypeStruct((NPAD, D), _f32),
                   jax.ShapeDtypeStruct((NPAD, 1), _f32)],
    )(x_pad, W0, d0, d1)


def _mid_call(a0, a1, xws, dinv, W):
    return pl.pallas_call(
        _mid_body,
        grid=(_NBLK,),
        in_specs=[
            pl.BlockSpec((BLK, D), _blk),
            pl.BlockSpec((BLK, D), _blk),
            pl.BlockSpec((BLK, D), _blk),
            pl.BlockSpec((BLK, 1), _blk),
            pl.BlockSpec((D, D), _fix),
        ],
        out_specs=pl.BlockSpec((BLK, D), _blk),
        out_shape=jax.ShapeDtypeStruct((NPAD, D), _f32),
    )(a0, a1, xws, dinv, W)


def _pool_call(a0, a1, xws, dinv, batch_pad):
    return pl.pallas_call(
        _pool_body,
        grid=(_NBLK,),
        in_specs=[
            pl.BlockSpec((BLK, D), _blk),
            pl.BlockSpec((BLK, D), _blk),
            pl.BlockSpec((BLK, D), _blk),
            pl.BlockSpec((BLK, 1), _blk),
            pl.BlockSpec((BLK, 1), _blk),
        ],
        out_specs=pl.BlockSpec((G, D), _fix),
        out_shape=jax.ShapeDtypeStruct((G, D), _f32),
        scratch_shapes=[pltpu.VMEM((G, D), _f32)],
    )(a0, a1, xws, dinv, batch_pad)


def _mlp_call(pool, batch_pad, lin1_W, lin1_b, bn_gamma, bn_beta, lin2_W, lin2_b):
    full = lambda s: pl.BlockSpec(s, _fix)
    return pl.pallas_call(
        _mlp_body,
        grid=(1,),
        in_specs=[
            full((G, D)), full((NPAD, 1)), full((D, D)), full((1, D)),
            full((1, D)), full((1, D)), full((D, O)), full((1, O)),
        ],
        out_specs=[full((G, O)), full((G, O)), full((G, O))],
        out_shape=[jax.ShapeDtypeStruct((G, O), _f32)] * 3,
    )(pool, batch_pad, lin1_W, lin1_b, bn_gamma, bn_beta, lin2_W, lin2_b)


def kernel(x, edge_index, edge_weight, batch, W0, W1, W2,
           lin1_W, lin1_b, bn_gamma, bn_beta, lin2_W, lin2_b):
    del edge_weight
    i32 = jnp.int32
    pad_e = jnp.full((EPAD - E,), N, dtype=i32)
    src_p = jnp.concatenate([edge_index[0], pad_e])
    dst_p = jnp.concatenate([edge_index[1], pad_e])
    comb_t = (src_p | (dst_p << 16)).reshape(NC, NS, CPT * CHUNK)
    x_pad = jnp.concatenate([x, jnp.zeros((NPAD - N, D), _f32)], axis=0)
    batch_pad = jnp.concatenate([batch, jnp.full((NPAD - N,), G, dtype=i32)])
    batch_pad = batch_pad.reshape(NPAD, 1)

    degp = _deg_kernel(comb_t)
    d0 = degp[0].reshape(NPAD, 1)
    d1 = degp[1].reshape(NPAD, 1)

    xws, dinv = _pre_call(x_pad, W0, d0, d1)

    accp = _edge_kernel(xws, comb_t)
    xws = _mid_call(accp[0], accp[1], xws, dinv, W1)

    accp = _edge_kernel(xws, comb_t)
    xws = _mid_call(accp[0], accp[1], xws, dinv, W2)

    accp = _edge_kernel(xws, comb_t)
    pool = _pool_call(accp[0], accp[1], xws, dinv, batch_pad)

    x_log, x_sig, last = _mlp_call(
        pool, batch_pad, lin1_W, lin1_b.reshape(1, D), bn_gamma.reshape(1, D),
        bn_beta.reshape(1, D), lin2_W, lin2_b.reshape(1, O))
    return (x_log, x_sig, last)


# speedup vs baseline: 1.0206x; 1.0206x over previous
import functools

import jax
import jax.numpy as jnp
from jax import lax
from jax.experimental import pallas as pl
from jax.experimental.pallas import tpu as pltpu
from jax.experimental.pallas import tpu_sc as plsc

N = 10000
D = 128
G = 128
O = 64
E = 320000

NC = 2
NS = 16
CHUNK = 96
CPT = 108
NBUF = 3
EPAD = NC * NS * CPT * CHUNK
NPAD = 10240
RPT = NPAD // NS

_mesh = plsc.VectorSubcoreMesh(core_axis_name="c", subcore_axis_name="s")


def _zero_vmem2d(ref, rows, cols):
    def body(i, _):
        for j in range(cols // 16):
            ref[i, pl.ds(j * 16, 16)] = jnp.zeros((16,), jnp.float32)
        return 0
    lax.fori_loop(0, rows, body, 0)


def _unpack(comb, j, iu, dst_only=False):
    for k in range(CHUNK // 16):
        v = comb[pl.ds(j * CHUNK + k * 16, 16)]
        if not dst_only:
            iu[0, pl.ds(k * 16, 16)] = v & jnp.int32(0xFFFF)
        iu[1, pl.ds(k * 16, 16)] = lax.shift_right_logical(v, 16)


@functools.partial(
    pl.kernel,
    out_type=jax.ShapeDtypeStruct((NC, NPAD), jnp.float32),
    mesh=_mesh,
    scratch_types=[
        pltpu.VMEM_SHARED((NPAD,), jnp.float32),
        pltpu.VMEM((CPT * CHUNK,), jnp.int32),
        pltpu.VMEM((2, CHUNK), jnp.int32),
        pltpu.VMEM((CHUNK,), jnp.float32),
        pltpu.VMEM((RPT,), jnp.float32),
    ],
)
def _deg_kernel(comb_t, degp, deg_sh, comb_v, iu, ones_v, zv):
    c = lax.axis_index("c")
    s = lax.axis_index("s")
    pltpu.sync_copy(comb_t.at[c, s], comb_v)

    def zbody(i, _):
        zv[pl.ds(i * 16, 16)] = jnp.zeros((16,), jnp.float32)
        return 0
    lax.fori_loop(0, RPT // 16, zbody, 0)

    def obody(i, _):
        ones_v[pl.ds(i * 16, 16)] = jnp.ones((16,), jnp.float32)
        return 0
    lax.fori_loop(0, CHUNK // 16, obody, 0)

    pltpu.sync_copy(zv, deg_sh.at[pl.ds(s * RPT, RPT)])
    plsc.subcore_barrier()

    def sbody(j, _):
        _unpack(comb_v, j, iu, dst_only=True)
        pltpu.sync_copy(ones_v, deg_sh.at[iu.at[1]], add=True)
        return 0
    lax.fori_loop(0, CPT, sbody, 0)

    plsc.subcore_barrier()
    pltpu.sync_copy(deg_sh.at[pl.ds(s * RPT, RPT)], degp.at[c, pl.ds(s * RPT, RPT)])


@functools.partial(
    pl.kernel,
    out_type=jax.ShapeDtypeStruct((NC, NPAD, D), jnp.float32),
    mesh=_mesh,
    scratch_types=[
        pltpu.VMEM_SHARED((NPAD, D), jnp.float32),
        pltpu.VMEM((CPT * CHUNK,), jnp.int32),
        pltpu.VMEM((2, CHUNK), jnp.int32),
        pltpu.VMEM((2, CHUNK), jnp.int32),
        pltpu.VMEM((CHUNK, D), jnp.float32),
        pltpu.VMEM((CHUNK, D), jnp.float32),
        pltpu.SemaphoreType.DMA,
        pltpu.SemaphoreType.DMA,
    ],
)
def _edge_kernel(xws, comb_t, accp,
                 acc_sh, comb_v, iu0, iu1, rows0, rows1, gsem0, gsem1):
    c = lax.axis_index("c")
    s = lax.axis_index("s")
    pltpu.sync_copy(comb_t.at[c, s], comb_v)

    _zero_vmem2d(rows0, 64, D)
    for r in range(RPT // 64):
        pltpu.sync_copy(rows0.at[pl.ds(0, 64)],
                        acc_sh.at[pl.ds(s * RPT + r * 64, 64)])
    plsc.subcore_barrier()

    def _drain(sem, buf):
        pltpu.make_async_copy(xws.at[pl.ds(0, CHUNK)], buf, sem).wait()

    _unpack(comb_v, 0, iu0)
    pltpu.async_copy(xws.at[iu0.at[0]], rows0, gsem0)

    def body(i, _):
        j0 = 2 * i
        j1 = j0 + 1
        _unpack(comb_v, j1, iu1)
        pltpu.async_copy(xws.at[iu1.at[0]], rows1, gsem1)
        _drain(gsem0, rows0)
        pltpu.sync_copy(rows0, acc_sh.at[iu0.at[1]], add=True)

        @pl.when(i < CPT // 2 - 1)
        def _():
            _unpack(comb_v, j0 + 2, iu0)
            pltpu.async_copy(xws.at[iu0.at[0]], rows0, gsem0)

        _drain(gsem1, rows1)
        pltpu.sync_copy(rows1, acc_sh.at[iu1.at[1]], add=True)
        return 0

    lax.fori_loop(0, CPT // 2, body, 0)

    plsc.subcore_barrier()
    pltpu.sync_copy(acc_sh.at[pl.ds(s * RPT, RPT)],
                    accp.at[c, pl.ds(s * RPT, RPT)])


BLK = 1024
_NBLK = NPAD // BLK


def _pre_body(x_ref, w_ref, d0_ref, d1_ref, xws_ref, dinv_ref):
    dinv = lax.rsqrt(d0_ref[...] + d1_ref[...] + 1.0)
    xw = jnp.dot(x_ref[...], w_ref[...], preferred_element_type=jnp.float32)
    xws_ref[...] = xw * dinv
    dinv_ref[...] = dinv


def _mid_body(a0_ref, a1_ref, xws_ref, dinv_ref, w_ref, out_ref):
    dinv = dinv_ref[...]
    h = jnp.maximum((a0_ref[...] + a1_ref[...] + xws_ref[...]) * dinv, 0.0)
    out_ref[...] = jnp.dot(h, w_ref[...], preferred_element_type=jnp.float32) * dinv


def _pool_body(a0_ref, a1_ref, xws_ref, dinv_ref, b_ref, out_ref, pool_acc):
    i = pl.program_id(0)
    h = jnp.maximum((a0_ref[...] + a1_ref[...] + xws_ref[...]) * dinv_ref[...], 0.0)
    oh = (b_ref[...] == lax.broadcasted_iota(jnp.int32, (1, G), 1)).astype(jnp.float32)
    p = lax.dot_general(oh, h, (((0,), (0,)), ((), ())),
                        preferred_element_type=jnp.float32, precision=lax.Precision.HIGHEST)

    @pl.when(i == 0)
    def _():
        pool_acc[...] = p

    @pl.when(i > 0)
    def _():
        pool_acc[...] = pool_acc[...] + p

    @pl.when(i == _NBLK - 1)
    def _():
        out_ref[...] = pool_acc[...]


def _mlp_body(pool_ref, b_ref, w1_ref, b1_ref, g_ref, be_ref, w2_ref, b2_ref,
              log_ref, sig_ref, last_ref):
    eq = (b_ref[...] == lax.broadcasted_iota(jnp.int32, (1, G), 1)).astype(jnp.float32)
    cnt = lax.dot_general(eq, jnp.ones((NPAD, 1), jnp.float32),
                          (((0,), (0,)), ((), ())),
                          preferred_element_type=jnp.float32, precision=lax.Precision.HIGHEST)
    g = pool_ref[...] / jnp.maximum(cnt, 1.0)
    m = jnp.dot(g, w1_ref[...], preferred_element_type=jnp.float32) + b1_ref[...]
    mu = jnp.mean(m, axis=0, keepdims=True)
    var = jnp.mean((m - mu) ** 2, axis=0, keepdims=True)
    m = (m - mu) / jnp.sqrt(var + 1e-5) * g_ref[...] + be_ref[...]
    m = jnp.maximum(m, 0.0)
    out = jnp.dot(m, w2_ref[...], preferred_element_type=jnp.float32) + b2_ref[...]
    xmax = jnp.max(out, axis=-1, keepdims=True)
    ex = jnp.exp(out - xmax)
    lse = jnp.log(jnp.sum(ex, axis=-1, keepdims=True)) + xmax
    log_ref[...] = out - lse
    sig_ref[...] = 1.0 / (1.0 + jnp.exp(-out))
    last_ref[...] = out


def _blk(i):
    return (i, 0)


def _fix(i):
    return (0, 0)


_f32 = jnp.float32


def _pre_call(x_pad, W0, d0, d1):
    return pl.pallas_call(
        _pre_body,
        grid=(_NBLK,),
        in_specs=[
            pl.BlockSpec((BLK, D), _blk),
            pl.BlockSpec((D, D), _fix),
            pl.BlockSpec((BLK, 1), _blk),
            pl.BlockSpec((BLK, 1), _blk),
        ],
        out_specs=[pl.BlockSpec((BLK, D), _blk), pl.BlockSpec((BLK, 1), _blk)],
        out_shape=[jax.ShapeDtypeStruct((NPAD, D), _f32),
                   jax.ShapeDtypeStruct((NPAD, 1), _f32)],
    )(x_pad, W0, d0, d1)


def _mid_call(a0, a1, xws, dinv, W):
    return pl.pallas_call(
        _mid_body,
        grid=(_NBLK,),
        in_specs=[
            pl.BlockSpec((BLK, D), _blk),
            pl.BlockSpec((BLK, D), _blk),
            pl.BlockSpec((BLK, D), _blk),
            pl.BlockSpec((BLK, 1), _blk),
            pl.BlockSpec((D, D), _fix),
        ],
        out_specs=pl.BlockSpec((BLK, D), _blk),
        out_shape=jax.ShapeDtypeStruct((NPAD, D), _f32),
    )(a0, a1, xws, dinv, W)


def _pool_call(a0, a1, xws, dinv, batch_pad):
    return pl.pallas_call(
        _pool_body,
        grid=(_NBLK,),
        in_specs=[
            pl.BlockSpec((BLK, D), _blk),
            pl.BlockSpec((BLK, D), _blk),
            pl.BlockSpec((BLK, D), _blk),
            pl.BlockSpec((BLK, 1), _blk),
            pl.BlockSpec((BLK, 1), _blk),
        ],
        out_specs=pl.BlockSpec((G, D), _fix),
        out_shape=jax.ShapeDtypeStruct((G, D), _f32),
        scratch_shapes=[pltpu.VMEM((G, D), _f32)],
    )(a0, a1, xws, dinv, batch_pad)


def _mlp_call(pool, batch_pad, lin1_W, lin1_b, bn_gamma, bn_beta, lin2_W, lin2_b):
    full = lambda s: pl.BlockSpec(s, _fix)
    return pl.pallas_call(
        _mlp_body,
        grid=(1,),
        in_specs=[
            full((G, D)), full((NPAD, 1)), full((D, D)), full((1, D)),
            full((1, D)), full((1, D)), full((D, O)), full((1, O)),
        ],
        out_specs=[full((G, O)), full((G, O)), full((G, O))],
        out_shape=[jax.ShapeDtypeStruct((G, O), _f32)] * 3,
    )(pool, batch_pad, lin1_W, lin1_b, bn_gamma, bn_beta, lin2_W, lin2_b)


def kernel(x, edge_index, edge_weight, batch, W0, W1, W2,
           lin1_W, lin1_b, bn_gamma, bn_beta, lin2_W, lin2_b):
    del edge_weight
    i32 = jnp.int32
    pad_e = jnp.full((EPAD - E,), N, dtype=i32)
    src_p = jnp.concatenate([edge_index[0], pad_e])
    dst_p = jnp.concatenate([edge_index[1], pad_e])
    comb_t = (src_p | (dst_p << 16)).reshape(NC, NS, CPT * CHUNK)
    x_pad = jnp.concatenate([x, jnp.zeros((NPAD - N, D), _f32)], axis=0)
    batch_pad = jnp.concatenate([batch, jnp.full((NPAD - N,), G, dtype=i32)])
    batch_pad = batch_pad.reshape(NPAD, 1)

    degp = _deg_kernel(comb_t)
    d0 = degp[0].reshape(NPAD, 1)
    d1 = degp[1].reshape(NPAD, 1)

    xws, dinv = _pre_call(x_pad, W0, d0, d1)

    accp = _edge_kernel(xws, comb_t)
    xws = _mid_call(accp[0], accp[1], xws, dinv, W1)

    accp = _edge_kernel(xws, comb_t)
    xws = _mid_call(accp[0], accp[1], xws, dinv, W2)

    accp = _edge_kernel(xws, comb_t)
    pool = _pool_call(accp[0], accp[1], xws, dinv, batch_pad)

    x_log, x_sig, last = _mlp_call(
        pool, batch_pad, lin1_W, lin1_b.reshape(1, D), bn_gamma.reshape(1, D),
        bn_beta.reshape(1, D), lin2_W, lin2_b.reshape(1, O))
    return (x_log, x_sig, last)


# speedup vs baseline: 1.4700x; 1.4403x over previous
import functools

import jax
import jax.numpy as jnp
from jax import lax
from jax.experimental import pallas as pl
from jax.experimental.pallas import tpu as pltpu
from jax.experimental.pallas import tpu_sc as plsc

N = 10000
D = 128
G = 128
O = 64
E = 320000

NC = 2
NS = 16
CHUNK = 96
CPT = 106
EPAD = NC * NS * CPT * CHUNK
NPAD = 10240
RPT = NPAD // NS

_mesh = plsc.VectorSubcoreMesh(core_axis_name="c", subcore_axis_name="s")


def _zero_vmem2d(ref, rows, cols):
    def body(i, _):
        for j in range(cols // 16):
            ref[i, pl.ds(j * 16, 16)] = jnp.zeros((16,), jnp.float32)
        return 0
    lax.fori_loop(0, rows, body, 0)


def _unpack(comb, j, iu, dst_only=False):
    for k in range(CHUNK // 16):
        v = comb[j, pl.ds(k * 16, 16)]
        if not dst_only:
            iu[0, pl.ds(k * 16, 16)] = v & jnp.int32(0xFFFF)
        iu[1, pl.ds(k * 16, 16)] = lax.shift_right_logical(v, 16)


@functools.partial(
    pl.kernel,
    out_type=jax.ShapeDtypeStruct((NC, NPAD), jnp.float32),
    mesh=_mesh,
    scratch_types=[
        pltpu.VMEM_SHARED((NPAD,), jnp.float32),
        pltpu.VMEM((CPT, CHUNK), jnp.int32),
        pltpu.VMEM((2, CHUNK), jnp.int32),
        pltpu.VMEM((CHUNK,), jnp.float32),
        pltpu.VMEM((RPT,), jnp.float32),
    ],
)
def _deg_kernel(comb_t, degp, deg_sh, comb_v, iu, ones_v, zv):
    c = lax.axis_index("c")
    s = lax.axis_index("s")
    pltpu.sync_copy(comb_t.at[c, s], comb_v)

    def zbody(i, _):
        zv[pl.ds(i * 16, 16)] = jnp.zeros((16,), jnp.float32)
        return 0
    lax.fori_loop(0, RPT // 16, zbody, 0)

    def obody(i, _):
        ones_v[pl.ds(i * 16, 16)] = jnp.ones((16,), jnp.float32)
        return 0
    lax.fori_loop(0, CHUNK // 16, obody, 0)

    pltpu.sync_copy(zv, deg_sh.at[pl.ds(s * RPT, RPT)])
    plsc.subcore_barrier()

    def sbody(j, _):
        _unpack(comb_v, j, iu, dst_only=True)
        pltpu.sync_copy(ones_v, deg_sh.at[iu.at[1]], add=True)
        return 0
    lax.fori_loop(0, CPT, sbody, 0)

    plsc.subcore_barrier()
    pltpu.sync_copy(deg_sh.at[pl.ds(s * RPT, RPT)], degp.at[c, pl.ds(s * RPT, RPT)])


@functools.partial(
    pl.kernel,
    out_type=jax.ShapeDtypeStruct((NC, NPAD, D), jnp.float32),
    mesh=_mesh,
    scratch_types=[
        pltpu.VMEM_SHARED((NPAD, D), jnp.float32),
        pltpu.VMEM((CPT, CHUNK), jnp.int32),
        pltpu.VMEM((2, CHUNK), jnp.int32),
        pltpu.VMEM((2, CHUNK), jnp.int32),
        pltpu.VMEM((CHUNK, D), jnp.float32),
        pltpu.VMEM((CHUNK, D), jnp.float32),
        pltpu.SemaphoreType.DMA,
        pltpu.SemaphoreType.DMA,
    ],
)
def _edge_kernel(xws, comb_t, accp,
                 acc_sh, comb_v, iu0, iu1, rows0, rows1, gsem0, gsem1):
    c = lax.axis_index("c")
    s = lax.axis_index("s")
    pltpu.sync_copy(comb_t.at[c, s], comb_v)

    _zero_vmem2d(rows0, 64, D)
    for r in range(RPT // 64):
        pltpu.sync_copy(rows0.at[pl.ds(0, 64)],
                        acc_sh.at[pl.ds(s * RPT + r * 64, 64)])
    plsc.subcore_barrier()

    def _drain(sem, buf):
        pltpu.make_async_copy(xws.at[pl.ds(0, CHUNK)], buf, sem).wait()

    _unpack(comb_v, 0, iu0)
    pltpu.async_copy(xws.at[iu0.at[0]], rows0, gsem0)

    def body(i, _):
        j0 = 2 * i
        j1 = j0 + 1
        _unpack(comb_v, j1, iu1)
        pltpu.async_copy(xws.at[iu1.at[0]], rows1, gsem1)
        _drain(gsem0, rows0)
        pltpu.sync_copy(rows0, acc_sh.at[iu0.at[1]], add=True)

        @pl.when(i < CPT // 2 - 1)
        def _():
            _unpack(comb_v, j0 + 2, iu0)
            pltpu.async_copy(xws.at[iu0.at[0]], rows0, gsem0)

        _drain(gsem1, rows1)
        pltpu.sync_copy(rows1, acc_sh.at[iu1.at[1]], add=True)
        return 0

    lax.fori_loop(0, CPT // 2, body, 0)

    plsc.subcore_barrier()
    pltpu.sync_copy(acc_sh.at[pl.ds(s * RPT, RPT)],
                    accp.at[c, pl.ds(s * RPT, RPT)])


BLK = 1024
_NBLK = NPAD // BLK


def _pre_body(x_ref, w_ref, d0_ref, d1_ref, xws_ref, dinv_ref):
    dinv = lax.rsqrt(d0_ref[...] + d1_ref[...] + 1.0)
    xw = jnp.dot(x_ref[...], w_ref[...], preferred_element_type=jnp.float32)
    xws_ref[...] = xw * dinv
    dinv_ref[...] = dinv


def _mid_body(a0_ref, a1_ref, xws_ref, dinv_ref, w_ref, out_ref):
    dinv = dinv_ref[...]
    h = jnp.maximum((a0_ref[...] + a1_ref[...] + xws_ref[...]) * dinv, 0.0)
    out_ref[...] = jnp.dot(h, w_ref[...], preferred_element_type=jnp.float32) * dinv


def _pool_body(a0_ref, a1_ref, xws_ref, dinv_ref, b_ref, out_ref, pool_acc):
    i = pl.program_id(0)
    h = jnp.maximum((a0_ref[...] + a1_ref[...] + xws_ref[...]) * dinv_ref[...], 0.0)
    oh = (b_ref[...] == lax.broadcasted_iota(jnp.int32, (1, G), 1)).astype(jnp.float32)
    p = lax.dot_general(oh, h, (((0,), (0,)), ((), ())),
                        preferred_element_type=jnp.float32, precision=lax.Precision.HIGHEST)

    @pl.when(i == 0)
    def _():
        pool_acc[...] = p

    @pl.when(i > 0)
    def _():
        pool_acc[...] = pool_acc[...] + p

    @pl.when(i == _NBLK - 1)
    def _():
        out_ref[...] = pool_acc[...]


def _mlp_body(pool_ref, b_ref, w1_ref, b1_ref, g_ref, be_ref, w2_ref, b2_ref,
              log_ref, sig_ref, last_ref):
    eq = (b_ref[...] == lax.broadcasted_iota(jnp.int32, (1, G), 1)).astype(jnp.float32)
    cnt = lax.dot_general(eq, jnp.ones((NPAD, 1), jnp.float32),
                          (((0,), (0,)), ((), ())),
                          preferred_element_type=jnp.float32, precision=lax.Precision.HIGHEST)
    g = pool_ref[...] / jnp.maximum(cnt, 1.0)
    m = jnp.dot(g, w1_ref[...], preferred_element_type=jnp.float32) + b1_ref[...]
    mu = jnp.mean(m, axis=0, keepdims=True)
    var = jnp.mean((m - mu) ** 2, axis=0, keepdims=True)
    m = (m - mu) / jnp.sqrt(var + 1e-5) * g_ref[...] + be_ref[...]
    m = jnp.maximum(m, 0.0)
    out = jnp.dot(m, w2_ref[...], preferred_element_type=jnp.float32) + b2_ref[...]
    xmax = jnp.max(out, axis=-1, keepdims=True)
    ex = jnp.exp(out - xmax)
    lse = jnp.log(jnp.sum(ex, axis=-1, keepdims=True)) + xmax
    log_ref[...] = out - lse
    sig_ref[...] = 1.0 / (1.0 + jnp.exp(-out))
    last_ref[...] = out


def _blk(i):
    return (i, 0)


def _fix(i):
    return (0, 0)


_f32 = jnp.float32


def _pre_call(x_pad, W0, d0, d1):
    return pl.pallas_call(
        _pre_body,
        grid=(_NBLK,),
        in_specs=[
            pl.BlockSpec((BLK, D), _blk),
            pl.BlockSpec((D, D), _fix),
            pl.BlockSpec((BLK, 1), _blk),
            pl.BlockSpec((BLK, 1), _blk),
        ],
        out_specs=[pl.BlockSpec((BLK, D), _blk), pl.BlockSpec((BLK, 1), _blk)],
        out_shape=[jax.ShapeDtypeStruct((NPAD, D), _f32),
                   jax.ShapeDtypeStruct((NPAD, 1), _f32)],
    )(x_pad, W0, d0, d1)


def _mid_call(a0, a1, xws, dinv, W):
    return pl.pallas_call(
        _mid_body,
        grid=(_NBLK,),
        in_specs=[
            pl.BlockSpec((BLK, D), _blk),
            pl.BlockSpec((BLK, D), _blk),
            pl.BlockSpec((BLK, D), _blk),
            pl.BlockSpec((BLK, 1), _blk),
            pl.BlockSpec((D, D), _fix),
        ],
        out_specs=pl.BlockSpec((BLK, D), _blk),
        out_shape=jax.ShapeDtypeStruct((NPAD, D), _f32),
    )(a0, a1, xws, dinv, W)


def _pool_call(a0, a1, xws, dinv, batch_pad):
    return pl.pallas_call(
        _pool_body,
        grid=(_NBLK,),
        in_specs=[
            pl.BlockSpec((BLK, D), _blk),
            pl.BlockSpec((BLK, D), _blk),
            pl.BlockSpec((BLK, D), _blk),
            pl.BlockSpec((BLK, 1), _blk),
            pl.BlockSpec((BLK, 1), _blk),
        ],
        out_specs=pl.BlockSpec((G, D), _fix),
        out_shape=jax.ShapeDtypeStruct((G, D), _f32),
        scratch_shapes=[pltpu.VMEM((G, D), _f32)],
    )(a0, a1, xws, dinv, batch_pad)


def _mlp_call(pool, batch_pad, lin1_W, lin1_b, bn_gamma, bn_beta, lin2_W, lin2_b):
    full = lambda s: pl.BlockSpec(s, _fix)
    return pl.pallas_call(
        _mlp_body,
        grid=(1,),
        in_specs=[
            full((G, D)), full((NPAD, 1)), full((D, D)), full((1, D)),
            full((1, D)), full((1, D)), full((D, O)), full((1, O)),
        ],
        out_specs=[full((G, O)), full((G, O)), full((G, O))],
        out_shape=[jax.ShapeDtypeStruct((G, O), _f32)] * 3,
    )(pool, batch_pad, lin1_W, lin1_b, bn_gamma, bn_beta, lin2_W, lin2_b)


def kernel(x, edge_index, edge_weight, batch, W0, W1, W2,
           lin1_W, lin1_b, bn_gamma, bn_beta, lin2_W, lin2_b):
    del edge_weight
    i32 = jnp.int32
    pad_e = jnp.full((EPAD - E,), N, dtype=i32)
    src_p = jnp.concatenate([edge_index[0], pad_e])
    dst_p = jnp.concatenate([edge_index[1], pad_e])
    comb_t = (src_p | (dst_p << 16)).reshape(NC, NS, CPT, CHUNK)
    x_pad = jnp.concatenate([x, jnp.zeros((NPAD - N, D), _f32)], axis=0)
    batch_pad = jnp.concatenate([batch, jnp.full((NPAD - N,), G, dtype=i32)])
    batch_pad = batch_pad.reshape(NPAD, 1)

    degp = _deg_kernel(comb_t)
    d0 = degp[0].reshape(NPAD, 1)
    d1 = degp[1].reshape(NPAD, 1)

    xws, dinv = _pre_call(x_pad, W0, d0, d1)

    accp = _edge_kernel(xws, comb_t)
    xws = _mid_call(accp[0], accp[1], xws, dinv, W1)

    accp = _edge_kernel(xws, comb_t)
    xws = _mid_call(accp[0], accp[1], xws, dinv, W2)

    accp = _edge_kernel(xws, comb_t)
    pool = _pool_call(accp[0], accp[1], xws, dinv, batch_pad)

    x_log, x_sig, last = _mlp_call(
        pool, batch_pad, lin1_W, lin1_b.reshape(1, D), bn_gamma.reshape(1, D),
        bn_beta.reshape(1, D), lin2_W, lin2_b.reshape(1, O))
    return (x_log, x_sig, last)
